# baseline (speedup 1.0000x reference)
import jax
import jax.numpy as jnp
from jax.experimental import pallas as pl

_PATCH = 4
_IMG_C = 3
_NUM_C = 96
_H = 512
_W = 512
_N = 32

_ROW_BYTES = _W * _IMG_C
_PW = _PATCH * _IMG_C
_NPW = _W // _PATCH
_HB = 512
_PB = (_HB // _PATCH) * _NPW


def _emb_kernel(x_ref, w_ref, b_ref, o_ref):
    xb = x_ref[0]
    xb4 = xb.reshape(_HB // _PATCH, _PATCH, _ROW_BYTES)
    acc = None
    for dh in range(_PATCH):
        rows = xb4[:, dh, :]
        t = rows.T
        t3 = t.reshape(_NPW, _PW, _HB // _PATCH)
        part = jax.lax.dot_general(
            t3, w_ref[dh * _PW:(dh + 1) * _PW, :],
            ((( 1,), (0,)), ((), ())),
            preferred_element_type=jnp.float32)
        acc = part if acc is None else acc + part
    e = acc.transpose(1, 0, 2)
    o_ref[0] = e.reshape(_PB, _NUM_C) + b_ref[0]


def _idx_kernel(o_ref):
    o_ref[...] = jax.lax.broadcasted_iota(jnp.int32, o_ref.shape, 1)


def kernel(x, W, b):
    n, h, w, c = x.shape
    xr = x.reshape(n, h, w * c)
    grid = (n, h // _HB)
    emb = pl.pallas_call(
        _emb_kernel,
        grid=grid,
        in_specs=[
            pl.BlockSpec((1, _HB, w * c), lambda i, j: (i, j, 0)),
            pl.BlockSpec((_PATCH * _PW, _NUM_C), lambda i, j: (0, 0)),
            pl.BlockSpec((1, _NUM_C), lambda i, j: (0, 0)),
        ],
        out_specs=pl.BlockSpec((1, _PB, _NUM_C), lambda i, j: (i, j, 0)),
        out_shape=jax.ShapeDtypeStruct(
            (n, (h // _PATCH) * (w // _PATCH), _NUM_C), jnp.float32),
    )(xr, W, b.reshape(1, _NUM_C))

    npatches = (h // _PATCH) * (w // _PATCH)
    idx = pl.pallas_call(
        _idx_kernel,
        out_shape=jax.ShapeDtypeStruct((n, npatches), jnp.int32),
    )()
    return emb, idx


# speedup vs baseline: 3.4029x; 3.4029x over previous
import jax
import jax.numpy as jnp
from jax.experimental import pallas as pl

_PATCH = 4
_IMG_C = 3
_NUM_C = 96
_NPW = 128
_HB = 128
_NB = _HB // _PATCH
_PB = _NB * _NPW


def _emb_kernel(x_ref, wt_ref, b_ref, o_ref):
    pieces = []
    for c in range(_IMG_C):
        plane = x_ref[0, c]
        pt3 = plane.T.reshape(_NPW, _PATCH, _HB)
        for dw in range(_PATCH):
            st3 = pt3[:, dw, :].T.reshape(_NB, _PATCH, _NPW)
            for dh in range(_PATCH):
                pieces.append(st3[:, dh, :])
    p = jnp.stack(pieces, axis=0).reshape(48, _PB)
    e = jax.lax.dot_general(
        wt_ref[...], p, (((1,), (0,)), ((), ())),
        preferred_element_type=jnp.float32)
    o_ref[0] = e + b_ref[...]


def _idx_kernel(o_ref):
    o_ref[...] = jax.lax.broadcasted_iota(jnp.int32, o_ref.shape, 1)


def kernel(x, W, b):
    n, h, w, c = x.shape
    npatches = (h // _PATCH) * (w // _PATCH)
    xt = x.transpose(0, 3, 1, 2)
    wrt = W.reshape(_PATCH, _PATCH, _IMG_C, _NUM_C)
    wrt = wrt.transpose(2, 1, 0, 3).reshape(48, _NUM_C).T

    grid = (n, h // _HB)
    embt = pl.pallas_call(
        _emb_kernel,
        grid=grid,
        in_specs=[
            pl.BlockSpec((1, _IMG_C, _HB, w), lambda i, j: (i, 0, j, 0)),
            pl.BlockSpec((_NUM_C, 48), lambda i, j: (0, 0)),
            pl.BlockSpec((_NUM_C, 1), lambda i, j: (0, 0)),
        ],
        out_specs=pl.BlockSpec((1, _NUM_C, _PB), lambda i, j: (i, 0, j)),
        out_shape=jax.ShapeDtypeStruct((n, _NUM_C, npatches), jnp.float32),
    )(xt, wrt, b.reshape(_NUM_C, 1))
    emb = embt.transpose(0, 2, 1)

    idx = pl.pallas_call(
        _idx_kernel,
        out_shape=jax.ShapeDtypeStruct((n, npatches), jnp.int32),
    )()
    return emb, idx


# speedup vs baseline: 4.5864x; 1.3478x over previous
import functools

import jax
import jax.numpy as jnp
from jax import lax
from jax.experimental import pallas as pl
from jax.experimental.pallas import tpu as pltpu
from jax.experimental.pallas import tpu_sc as plsc

_PATCH = 4
_IMG_C = 3
_NUM_C = 96
_NPW = 128
_HB = 512
_NB = _HB // _PATCH
_PB = _NB * _NPW


def _emb_kernel(x_ref, s_ref, wt_ref, b_ref, o_ref):
    pieces = []
    for c in range(_IMG_C):
        plane = x_ref[0, c]
        r = jnp.dot(plane, s_ref[...],
                    preferred_element_type=jnp.float32)
        for dw in range(_PATCH):
            g3 = r[:, dw * _NPW:(dw + 1) * _NPW].reshape(_NB, _PATCH, _NPW)
            for dh in range(_PATCH):
                pieces.append(g3[:, dh, :])
    p = jnp.stack(pieces, axis=0).reshape(48, _PB)
    e = jax.lax.dot_general(
        wt_ref[...], p, (((1,), (0,)), ((), ())),
        preferred_element_type=jnp.float32)
    o_ref[0] = e + b_ref[...]


def _make_idx_sc(n, npatches):
    info = plsc.get_sparse_core_info()
    nc = info.num_cores
    lanes = info.num_lanes
    mesh = plsc.VectorSubcoreMesh(core_axis_name="c", subcore_axis_name="s")

    @functools.partial(
        pl.kernel, mesh=mesh,
        out_type=jax.ShapeDtypeStruct((n, npatches), jnp.int32),
        scratch_types=[pltpu.VMEM((npatches,), jnp.int32)],
    )
    def idx_kernel(out_hbm, ramp_v):
        wid = lax.axis_index("s") * nc + lax.axis_index("c")

        def body(k, carry):
            ramp_v[pl.ds(k * lanes, lanes)] = (
                lax.iota(jnp.int32, lanes) + k * lanes)
            return carry

        lax.fori_loop(0, npatches // lanes, body, 0)
        pltpu.sync_copy(ramp_v, out_hbm.at[wid])

    return idx_kernel


def kernel(x, W, b):
    n, h, w, c = x.shape
    npatches = (h // _PATCH) * (w // _PATCH)
    xt = x.transpose(0, 3, 1, 2)
    wrt = W.reshape(_PATCH, _PATCH, _IMG_C, _NUM_C)
    wrt = wrt.transpose(2, 1, 0, 3).reshape(48, _NUM_C).T
    wi = jnp.arange(w)[:, None]
    qi = jnp.arange(w)[None, :]
    sel = ((wi % _PATCH == qi // _NPW) & (wi // _PATCH == qi % _NPW))
    sel = sel.astype(jnp.float32)

    grid = (n, h // _HB)
    embt = pl.pallas_call(
        _emb_kernel,
        grid=grid,
        in_specs=[
            pl.BlockSpec((1, _IMG_C, _HB, w), lambda i, j: (i, 0, j, 0)),
            pl.BlockSpec((w, w), lambda i, j: (0, 0)),
            pl.BlockSpec((_NUM_C, 48), lambda i, j: (0, 0)),
            pl.BlockSpec((_NUM_C, 1), lambda i, j: (0, 0)),
        ],
        out_specs=pl.BlockSpec((1, _NUM_C, _PB), lambda i, j: (i, 0, j)),
        out_shape=jax.ShapeDtypeStruct((n, _NUM_C, npatches), jnp.float32),
    )(xt, sel, wrt, b.reshape(_NUM_C, 1))
    emb = embt.transpose(0, 2, 1)

    idx = _make_idx_sc(n, npatches)()
    return emb, idx


# speedup vs baseline: 5.0128x; 1.0930x over previous
import jax
import jax.numpy as jnp
from jax.experimental import pallas as pl

_PATCH = 4
_IMG_C = 3
_NUM_C = 96
_NPW = 128
_HB = 512
_NB = _HB // _PATCH
_PB = _NB * _NPW


def _emb_kernel(x_ref, s_ref, wt_ref, b_ref, o_ref):
    pieces = []
    for c in range(_IMG_C):
        plane = x_ref[0, c]
        r = jnp.dot(plane, s_ref[...],
                    preferred_element_type=jnp.float32)
        for dw in range(_PATCH):
            g3 = r[:, dw * _NPW:(dw + 1) * _NPW].reshape(_NB, _PATCH, _NPW)
            for dh in range(_PATCH):
                pieces.append(g3[:, dh, :])
    p = jnp.stack(pieces, axis=0).reshape(48, _PB)
    e = jax.lax.dot_general(
        wt_ref[...], p, (((1,), (0,)), ((), ())),
        preferred_element_type=jnp.float32)
    o_ref[0] = e + b_ref[...]


def _idx_kernel(o_ref):
    o_ref[...] = jax.lax.broadcasted_iota(jnp.int32, o_ref.shape, 1)


def kernel(x, W, b):
    n, h, w, c = x.shape
    npatches = (h // _PATCH) * (w // _PATCH)
    xt = x.transpose(0, 3, 1, 2)
    wrt = W.reshape(_PATCH, _PATCH, _IMG_C, _NUM_C)
    wrt = wrt.transpose(2, 1, 0, 3).reshape(48, _NUM_C).T
    wi = jnp.arange(w)[:, None]
    qi = jnp.arange(w)[None, :]
    sel = ((wi % _PATCH == qi // _NPW) & (wi // _PATCH == qi % _NPW))
    sel = sel.astype(jnp.float32)

    grid = (n, h // _HB)
    embt = pl.pallas_call(
        _emb_kernel,
        grid=grid,
        in_specs=[
            pl.BlockSpec((1, _IMG_C, _HB, w), lambda i, j: (i, 0, j, 0)),
            pl.BlockSpec((w, w), lambda i, j: (0, 0)),
            pl.BlockSpec((_NUM_C, 48), lambda i, j: (0, 0)),
            pl.BlockSpec((_NUM_C, 1), lambda i, j: (0, 0)),
        ],
        out_specs=pl.BlockSpec((1, _NUM_C, _PB), lambda i, j: (i, 0, j)),
        out_shape=jax.ShapeDtypeStruct((n, _NUM_C, npatches), jnp.float32),
    )(xt, sel, wrt, b.reshape(_NUM_C, 1))
    emb = embt.transpose(0, 2, 1)

    idx = pl.pallas_call(
        _idx_kernel,
        out_shape=jax.ShapeDtypeStruct((n, npatches), jnp.int32),
    )()
    return emb, idx
